# baseline (speedup 1.0000x reference)
import jax
import jax.numpy as jnp
from jax.experimental import pallas as pl
from jax.experimental.pallas import tpu as pltpu

_BR = 2000
_Q = 2
_SPLIT = 384


def _fused_mlp_kernel(x_ref, w1_ref, b1_ref, w2_ref, b2_ref, o_hbm,
                      main_buf, tail_buf, main_sems, tail_sems):
    i = pl.program_id(0)
    n = pl.num_programs(0)
    slot = jax.lax.rem(i, _Q)
    tail_w = tail_buf.shape[2]

    def main_copy(s, j):
        return pltpu.make_async_copy(
            main_buf.at[s],
            o_hbm.at[pl.ds(j * _BR, _BR), pl.ds(0, _SPLIT)],
            main_sems.at[s],
        )

    def tail_copy(s, j):
        return pltpu.make_async_copy(
            tail_buf.at[s],
            o_hbm.at[pl.ds(j * _BR, _BR), pl.ds(_SPLIT, tail_w)],
            tail_sems.at[s],
        )

    @pl.when(i >= _Q)
    def _wait_prev():
        main_copy(slot, i - _Q).wait()
        tail_copy(slot, i - _Q).wait()

    h = jax.lax.dot_general(
        x_ref[...], w1_ref[...],
        dimension_numbers=(((1,), (0,)), ((), ())),
        preferred_element_type=jnp.float32,
    )
    h = jnp.maximum(h + b1_ref[...], 0.0)
    o = jax.lax.dot_general(
        h, w2_ref[...],
        dimension_numbers=(((1,), (0,)), ((), ())),
        preferred_element_type=jnp.float32,
    )
    o = o + b2_ref[...]
    main_buf[slot] = o[:, :_SPLIT]
    tail_buf[slot] = o[:, _SPLIT:]

    main_copy(slot, i).start()
    tail_copy(slot, i).start()

    @pl.when(i == n - 1)
    def _drain():
        for q in range(1, _Q + 1):
            j = n - 1 - _Q + q
            s = jax.lax.rem(jnp.int32(j), _Q)
            main_copy(s, j).wait()
            tail_copy(s, j).wait()


def kernel(x_transaction, x_merchant, x_category, edge_index_belongs_to, edge_index_has_category, W_enc_t, b_enc_t, W_enc_m, b_enc_m, W_enc_c, b_enc_c, lin_l_bm_0, bias_bm_0, lin_r_bm_0, lin_l_tc_0, bias_tc_0, lin_r_tc_0, lin_l_bm_1, bias_bm_1, lin_r_bm_1, lin_l_tc_1, bias_tc_1, lin_r_tc_1, W_cls, b_cls):
    NT, D = x_transaction.shape
    H = W_enc_t.shape[1]
    OUT = W_cls.shape[1]

    grid = (NT // _BR,)

    b1 = b_enc_t.reshape(1, H)
    b2 = b_cls.reshape(1, OUT)

    return pl.pallas_call(
        _fused_mlp_kernel,
        grid=grid,
        in_specs=[
            pl.BlockSpec((_BR, D), lambda i: (i, 0)),
            pl.BlockSpec((D, H), lambda i: (0, 0)),
            pl.BlockSpec((1, H), lambda i: (0, 0)),
            pl.BlockSpec((H, OUT), lambda i: (0, 0)),
            pl.BlockSpec((1, OUT), lambda i: (0, 0)),
        ],
        out_specs=pl.BlockSpec(memory_space=pltpu.MemorySpace.HBM),
        out_shape=jax.ShapeDtypeStruct((NT, OUT), jnp.float32),
        scratch_shapes=[
            pltpu.VMEM((_Q, _BR, _SPLIT), jnp.float32),
            pltpu.VMEM((_Q, _BR, OUT - _SPLIT), jnp.float32),
            pltpu.SemaphoreType.DMA((_Q,)),
            pltpu.SemaphoreType.DMA((_Q,)),
        ],
    )(x_transaction, W_enc_t, b1, W_cls, b2)
